# baseline (speedup 1.0000x reference)
import functools

import numpy as np
import jax
import jax.numpy as jnp
from jax import lax
from jax.experimental import pallas as pl
from jax.experimental.pallas import tpu as pltpu
from jax.experimental.pallas import tpu_sc as plsc

NCAT = 26
HASH = 100000
DM = 32
BATCH = 4096
NPAIR = NCAT * (NCAT - 1) // 2

_info = plsc.get_sparse_core_info()
_NC, _NS, _L = _info.num_cores, _info.num_subcores, _info.num_lanes
NW = _NC * _NS

BPW = BATCH // NW
PG = 5
NBLK = NPAIR // PG
CROWS = PG * BPW
NTD = DM // 8

IPAD = 328

_i_np, _j_np = np.triu_indices(NCAT, k=1)
_II = np.zeros((IPAD,), np.int32)
_JJ = np.zeros((IPAD,), np.int32)
_II[:NPAIR] = _i_np
_JJ[:NPAIR] = _j_np


def _mod_const(n, d):
    rinv = np.float32(1.0) / np.float32(d)
    q = (n.astype(jnp.float32) * rinv).astype(jnp.int32)
    r = n - q * d
    q = jnp.where(r >= d, q + 1, q)
    q = jnp.where(r < 0, q - 1, q)
    return n - q * d


def _sc_body(cat_hbm, tab_hbm, ii_hbm, jj_hbm, out_hbm,
             cat_v, ii_v, jj_v, idx0, idx1, idx2, rows0, rows1, rows2,
             stg0, stg1, gsem, wsem):
    wid = lax.axis_index("s") * _NC + lax.axis_index("c")
    pltpu.sync_copy(cat_hbm.at[pl.ds(wid * (BPW * NCAT), BPW * NCAT)], cat_v)
    pltpu.sync_copy(ii_hbm, ii_v)
    pltpu.sync_copy(jj_hbm, jj_v)
    lanes = lax.iota(jnp.int32, _L)

    def comp_fire(blk, idx_v, rows_v):
        p0 = blk * PG

        def comp(pl_i, c):
            pvec = (p0 + pl_i) + lanes * 0
            ip = plsc.load_gather(ii_v, [pvec])
            jp = plsc.load_gather(jj_v, [pvec])
            for v in range(BPW // _L):
                b_l = v * _L + lanes
                base26 = b_l * NCAT
                ci = plsc.load_gather(cat_v, [base26 + ip])
                cj = plsc.load_gather(cat_v, [base26 + jp])
                h = _mod_const(ci * 17 + cj * 31, HASH)
                idx_v[pl.ds(pl_i * BPW + v * _L, _L)] = h
            return c
        lax.fori_loop(0, PG, comp, 0)

        def fire(si, c):
            pltpu.async_copy(tab_hbm.at[idx_v.at[pl.ds(si * 128, 128)]],
                             rows_v.at[pl.ds(si * 128, 128)], gsem)
            return c
        lax.fori_loop(0, PG, fire, 0)

    def wfire(blk, stg_v):
        def per_p(pl_i, c):
            pltpu.async_copy(
                stg_v.at[pl.ds(pl_i * NTD, NTD), :, pl.ds(0, BPW)],
                out_hbm.at[pl.ds((blk * PG + pl_i) * NTD, NTD),
                           pl.ds(wid * 8, 8)], wsem)
            return c
        lax.fori_loop(0, PG, per_p, 0)

    def wdrain():
        pltpu.make_async_copy(out_hbm.at[pl.ds(0, PG * NTD), pl.ds(0, 8)],
                              stg0.at[:, :, pl.ds(0, BPW)], wsem).wait()

    lanes_td = lax.shift_right_logical(lanes, 3)
    lanes_s = lanes & 7

    def process(blk, idx_v, rows_v, stg_v):
        @pl.when(blk > 1)
        def _():
            wdrain()

        pltpu.make_async_copy(out_hbm.at[pl.ds(0, PG * NTD), pl.ds(0, 8)],
                              stg0.at[:, :, pl.ds(0, BPW)], gsem).wait()

        def trans_p(pl_i, c):
            td0 = pl_i * NTD + lanes_td
            td1 = td0 + 2

            def trans_g(g, c2):
                xs = []
                for j in range(16):
                    r = pl_i * BPW + g * 16 + j
                    xs.append((rows_v[r, pl.ds(0, _L)],
                               rows_v[r, pl.ds(_L, _L)]))
                for j, (x0, x1) in enumerate(xs):
                    cvec = (g * 16 + j) + lanes * 0
                    plsc.store_scatter(stg_v, [td0, lanes_s, cvec], x0)
                    plsc.store_scatter(stg_v, [td1, lanes_s, cvec], x1)
                return c2
            lax.fori_loop(0, BPW // 16, trans_g, 0)
            return c
        lax.fori_loop(0, PG, trans_p, 0)

        wfire(blk, stg_v)

    idxs = (idx0, idx1, idx2)
    rows = (rows0, rows1, rows2)
    stgs = (stg0, stg1)
    comp_fire(0, idx0, rows0)
    comp_fire(1, idx1, rows1)

    def step(t, carry):
        b = 6 * t
        for i in range(6):
            comp_fire(b + 2 + i, idxs[(2 + i) % 3], rows[(2 + i) % 3])
            process(b + i, idxs[i % 3], rows[i % 3], stgs[i % 2])
        return carry

    lax.fori_loop(0, 10, step, 0)
    comp_fire(62, idx2, rows2)
    process(60, idx0, rows0, stg0)
    comp_fire(63, idx0, rows0)
    process(61, idx1, rows1, stg1)
    comp_fire(64, idx1, rows1)
    process(62, idx2, rows2, stg0)
    process(63, idx0, rows0, stg1)
    process(64, idx1, rows1, stg0)
    wdrain()
    wdrain()


_mesh = plsc.VectorSubcoreMesh(core_axis_name="c", subcore_axis_name="s")

_sc_kernel = functools.partial(
    pl.kernel,
    mesh=_mesh,
    out_type=jax.ShapeDtypeStruct((NPAIR * NTD, NW * 8, 128), jnp.float32),
    scratch_types=[
        pltpu.VMEM((BPW * NCAT,), jnp.int32),
        pltpu.VMEM((IPAD,), jnp.int32),
        pltpu.VMEM((IPAD,), jnp.int32),
        pltpu.VMEM((CROWS,), jnp.int32),
        pltpu.VMEM((CROWS,), jnp.int32),
        pltpu.VMEM((CROWS,), jnp.int32),
        pltpu.VMEM((CROWS, DM), jnp.float32),
        pltpu.VMEM((CROWS, DM), jnp.float32),
        pltpu.VMEM((CROWS, DM), jnp.float32),
        pltpu.VMEM((PG * NTD, 8, BPW + 1), jnp.float32),
        pltpu.VMEM((PG * NTD, 8, BPW + 1), jnp.float32),
        pltpu.SemaphoreType.DMA,
        pltpu.SemaphoreType.DMA,
    ],
    compiler_params=pltpu.CompilerParams(use_tc_tiling_on_sc=False,
                                         needs_layout_passes=False),
)(_sc_body)


def kernel(cat_features, interaction_table):
    cat_flat = cat_features.reshape(-1)
    ii = jnp.asarray(_II)
    jj = jnp.asarray(_JJ)
    out = _sc_kernel(cat_flat, interaction_table, ii, jj)
    out5 = out.reshape(NPAIR, NTD, NW, 8, 128)
    return out5.transpose(2, 4, 0, 1, 3).reshape(BATCH, NPAIR, DM)


# speedup vs baseline: 1.0143x; 1.0143x over previous
import functools

import numpy as np
import jax
import jax.numpy as jnp
from jax import lax
from jax.experimental import pallas as pl
from jax.experimental.pallas import tpu as pltpu
from jax.experimental.pallas import tpu_sc as plsc

NCAT = 26
HASH = 100000
DM = 32
BATCH = 4096
NPAIR = NCAT * (NCAT - 1) // 2

_info = plsc.get_sparse_core_info()
_NC, _NS, _L = _info.num_cores, _info.num_subcores, _info.num_lanes
NW = _NC * _NS

BPW = BATCH // NW
PG = 5
NBLK = NPAIR // PG
CROWS = PG * BPW
NTD = DM // 8

IPAD = 328

_i_np, _j_np = np.triu_indices(NCAT, k=1)
_II = np.zeros((IPAD,), np.int32)
_JJ = np.zeros((IPAD,), np.int32)
_II[:NPAIR] = _i_np
_JJ[:NPAIR] = _j_np


def _mod_const(n, d):
    rinv = np.float32(1.0) / np.float32(d)
    q = (n.astype(jnp.float32) * rinv).astype(jnp.int32)
    r = n - q * d
    q = jnp.where(r >= d, q + 1, q)
    q = jnp.where(r < 0, q - 1, q)
    return n - q * d


def _sc_body(cat_hbm, tab_hbm, ii_hbm, jj_hbm, out_hbm,
             cat_v, ii_v, jj_v, idx0, idx1, idx2, rows0, rows1, rows2,
             stg_v, gsem, wsem):
    wid = lax.axis_index("s") * _NC + lax.axis_index("c")
    pltpu.sync_copy(cat_hbm.at[pl.ds(wid * (BPW * NCAT), BPW * NCAT)], cat_v)
    pltpu.sync_copy(ii_hbm, ii_v)
    pltpu.sync_copy(jj_hbm, jj_v)
    lanes = lax.iota(jnp.int32, _L)

    def comp_fire(blk, idx_v, rows_v):
        p0 = blk * PG

        def comp(pl_i, c):
            pvec = (p0 + pl_i) + lanes * 0
            ip = plsc.load_gather(ii_v, [pvec])
            jp = plsc.load_gather(jj_v, [pvec])
            for v in range(BPW // _L):
                b_l = v * _L + lanes
                base26 = b_l * NCAT
                ci = plsc.load_gather(cat_v, [base26 + ip])
                cj = plsc.load_gather(cat_v, [base26 + jp])
                h = _mod_const(ci * 17 + cj * 31, HASH)
                idx_v[pl.ds(pl_i * BPW + v * _L, _L)] = h
            return c
        lax.fori_loop(0, PG, comp, 0)

        def fire(si, c):
            pltpu.async_copy(tab_hbm.at[idx_v.at[pl.ds(si * 128, 128)]],
                             rows_v.at[pl.ds(si * 128, 128)], gsem)
            return c
        lax.fori_loop(0, PG, fire, 0)

    def wfire(blk):
        def per_p(pl_i, c):
            pltpu.async_copy(
                stg_v.at[pl.ds(pl_i * NTD, NTD), :, pl.ds(0, BPW)],
                out_hbm.at[pl.ds((blk * PG + pl_i) * NTD, NTD),
                           pl.ds(wid * 8, 8)], wsem)
            return c
        lax.fori_loop(0, PG, per_p, 0)

    def wdrain():
        pltpu.make_async_copy(out_hbm.at[pl.ds(0, PG * NTD), pl.ds(0, 8)],
                              stg_v.at[:, :, pl.ds(0, BPW)], wsem).wait()

    lanes_td = lax.shift_right_logical(lanes, 3)
    lanes_s = lanes & 7

    def process(blk, idx_v, rows_v):
        @pl.when(blk > 0)
        def _():
            wdrain()

        pltpu.make_async_copy(out_hbm.at[pl.ds(0, PG * NTD), pl.ds(0, 8)],
                              stg_v.at[:, :, pl.ds(0, BPW)], gsem).wait()

        def trans_p(pl_i, c):
            td0 = pl_i * NTD + lanes_td
            td1 = td0 + 2

            def trans_g(g, c2):
                xs = []
                for j in range(16):
                    r = pl_i * BPW + g * 16 + j
                    xs.append((rows_v[r, pl.ds(0, _L)],
                               rows_v[r, pl.ds(_L, _L)]))
                for j, (x0, x1) in enumerate(xs):
                    cvec = (g * 16 + j) + lanes * 0
                    plsc.store_scatter(stg_v, [td0, lanes_s, cvec], x0)
                    plsc.store_scatter(stg_v, [td1, lanes_s, cvec], x1)
                return c2
            lax.fori_loop(0, BPW // 16, trans_g, 0)
            return c
        lax.fori_loop(0, PG, trans_p, 0)

        wfire(blk)

    comp_fire(0, idx0, rows0)
    comp_fire(1, idx1, rows1)

    def step(t, carry):
        b = 3 * t
        comp_fire(b + 2, idx2, rows2)
        process(b, idx0, rows0)
        comp_fire(b + 3, idx0, rows0)
        process(b + 1, idx1, rows1)
        comp_fire(b + 4, idx1, rows1)
        process(b + 2, idx2, rows2)
        return carry

    lax.fori_loop(0, (NBLK - 2) // 3, step, 0)
    process(NBLK - 2, idx0, rows0)
    process(NBLK - 1, idx1, rows1)
    wdrain()


_mesh = plsc.VectorSubcoreMesh(core_axis_name="c", subcore_axis_name="s")

_sc_kernel = functools.partial(
    pl.kernel,
    mesh=_mesh,
    out_type=jax.ShapeDtypeStruct((NPAIR * NTD, NW * 8, 128), jnp.float32),
    scratch_types=[
        pltpu.VMEM((BPW * NCAT,), jnp.int32),
        pltpu.VMEM((IPAD,), jnp.int32),
        pltpu.VMEM((IPAD,), jnp.int32),
        pltpu.VMEM((CROWS,), jnp.int32),
        pltpu.VMEM((CROWS,), jnp.int32),
        pltpu.VMEM((CROWS,), jnp.int32),
        pltpu.VMEM((CROWS, DM), jnp.float32),
        pltpu.VMEM((CROWS, DM), jnp.float32),
        pltpu.VMEM((CROWS, DM), jnp.float32),
        pltpu.VMEM((PG * NTD, 8, BPW + 1), jnp.float32),
        pltpu.SemaphoreType.DMA,
        pltpu.SemaphoreType.DMA,
    ],
    compiler_params=pltpu.CompilerParams(use_tc_tiling_on_sc=False,
                                         needs_layout_passes=False),
)(_sc_body)


def kernel(cat_features, interaction_table):
    cat_flat = cat_features.reshape(-1)
    ii = jnp.asarray(_II)
    jj = jnp.asarray(_JJ)
    out = _sc_kernel(cat_flat, interaction_table, ii, jj)
    out5 = out.reshape(NPAIR, NTD, NW, 8, 128)
    return out5.transpose(2, 4, 0, 1, 3).reshape(BATCH, NPAIR, DM)


# speedup vs baseline: 1.0539x; 1.0390x over previous
import functools

import numpy as np
import jax
import jax.numpy as jnp
from jax import lax
from jax.experimental import pallas as pl
from jax.experimental.pallas import tpu as pltpu
from jax.experimental.pallas import tpu_sc as plsc

NCAT = 26
HASH = 100000
DM = 32
BATCH = 4096
NPAIR = NCAT * (NCAT - 1) // 2

_info = plsc.get_sparse_core_info()
_NC, _NS, _L = _info.num_cores, _info.num_subcores, _info.num_lanes
NW = _NC * _NS

BPW = BATCH // NW
PG = 5
NBLK = NPAIR // PG
CROWS = PG * BPW
NTD = DM // 8

IPAD = 328

_i_np, _j_np = np.triu_indices(NCAT, k=1)
_II = np.zeros((IPAD,), np.int32)
_JJ = np.zeros((IPAD,), np.int32)
_II[:NPAIR] = _i_np
_JJ[:NPAIR] = _j_np


def _mod_const(n, d):
    rinv = np.float32(1.0) / np.float32(d)
    q = (n.astype(jnp.float32) * rinv).astype(jnp.int32)
    r = n - q * d
    q = jnp.where(r >= d, q + 1, q)
    q = jnp.where(r < 0, q - 1, q)
    return n - q * d


def _sc_body(cat_hbm, tab_hbm, ii_hbm, jj_hbm, out_hbm,
             cat_v, ii_v, jj_v, idx0, idx1, idx2, rows0, rows1, rows2,
             stg_v, gsem, wsem):
    wid = lax.axis_index("s") * _NC + lax.axis_index("c")
    pltpu.sync_copy(cat_hbm.at[pl.ds(wid * (BPW * NCAT), BPW * NCAT)], cat_v)
    pltpu.sync_copy(ii_hbm, ii_v)
    pltpu.sync_copy(jj_hbm, jj_v)
    lanes = lax.iota(jnp.int32, _L)

    def comp_fire(blk, idx_v, rows_v):
        p0 = blk * PG

        def comp(pl_i, c):
            pvec = (p0 + pl_i) + lanes * 0
            ip = plsc.load_gather(ii_v, [pvec])
            jp = plsc.load_gather(jj_v, [pvec])
            for v in range(BPW // _L):
                b_l = v * _L + lanes
                base26 = b_l * NCAT
                ci = plsc.load_gather(cat_v, [base26 + ip])
                cj = plsc.load_gather(cat_v, [base26 + jp])
                h = _mod_const(ci * 17 + cj * 31, HASH)
                idx_v[pl.ds(pl_i * BPW + v * _L, _L)] = h
            return c
        lax.fori_loop(0, PG, comp, 0)

        def fire(si, c):
            pltpu.async_copy(tab_hbm.at[idx_v.at[pl.ds(si * 128, 128)]],
                             rows_v.at[pl.ds(si * 128, 128)], gsem)
            return c
        lax.fori_loop(0, PG, fire, 0)

    def wfire(blk):
        def per_p(pl_i, c):
            pltpu.async_copy(
                stg_v.at[pl.ds(pl_i * NTD, NTD), :, pl.ds(0, BPW)],
                out_hbm.at[pl.ds((blk * PG + pl_i) * NTD, NTD),
                           pl.ds(wid * 8, 8)], wsem)
            return c
        lax.fori_loop(0, PG, per_p, 0)

    def wdrain():
        pltpu.make_async_copy(out_hbm.at[pl.ds(0, PG * NTD), pl.ds(0, 8)],
                              stg_v.at[:, :, pl.ds(0, BPW)], wsem).wait()

    lanes_td = lax.shift_right_logical(lanes, 3)
    lanes_s = lanes & 7

    def process(blk, idx_v, rows_v):
        @pl.when(blk > 0)
        def _():
            wdrain()

        pltpu.make_async_copy(out_hbm.at[pl.ds(0, PG * NTD), pl.ds(0, 8)],
                              stg_v.at[:, :, pl.ds(0, BPW)], gsem).wait()

        def trans_p(pl_i, c):
            td0 = pl_i * NTD + lanes_td
            td1 = td0 + 2

            def trans_g(g, c2):
                base = pl_i * BPW + g * 16

                def load(j):
                    return (rows_v[base + j, pl.ds(0, _L)],
                            rows_v[base + j, pl.ds(_L, _L)])
                xs = [load(0), load(1)]
                for j in range(16):
                    if j + 2 < 16:
                        xs.append(load(j + 2))
                    x0, x1 = xs[j]
                    cvec = (g * 16 + j) + lanes * 0
                    plsc.store_scatter(stg_v, [td0, lanes_s, cvec], x0)
                    plsc.store_scatter(stg_v, [td1, lanes_s, cvec], x1)
                return c2
            lax.fori_loop(0, BPW // 16, trans_g, 0)
            return c
        lax.fori_loop(0, PG, trans_p, 0)

        wfire(blk)

    comp_fire(0, idx0, rows0)
    comp_fire(1, idx1, rows1)

    def step(t, carry):
        b = 3 * t
        comp_fire(b + 2, idx2, rows2)
        process(b, idx0, rows0)
        comp_fire(b + 3, idx0, rows0)
        process(b + 1, idx1, rows1)
        comp_fire(b + 4, idx1, rows1)
        process(b + 2, idx2, rows2)
        return carry

    lax.fori_loop(0, (NBLK - 2) // 3, step, 0)
    process(NBLK - 2, idx0, rows0)
    process(NBLK - 1, idx1, rows1)
    wdrain()


_mesh = plsc.VectorSubcoreMesh(core_axis_name="c", subcore_axis_name="s")

_sc_kernel = functools.partial(
    pl.kernel,
    mesh=_mesh,
    out_type=jax.ShapeDtypeStruct((NPAIR * NTD, NW * 8, 128), jnp.float32),
    scratch_types=[
        pltpu.VMEM((BPW * NCAT,), jnp.int32),
        pltpu.VMEM((IPAD,), jnp.int32),
        pltpu.VMEM((IPAD,), jnp.int32),
        pltpu.VMEM((CROWS,), jnp.int32),
        pltpu.VMEM((CROWS,), jnp.int32),
        pltpu.VMEM((CROWS,), jnp.int32),
        pltpu.VMEM((CROWS, DM), jnp.float32),
        pltpu.VMEM((CROWS, DM), jnp.float32),
        pltpu.VMEM((CROWS, DM), jnp.float32),
        pltpu.VMEM((PG * NTD, 8, BPW + 1), jnp.float32),
        pltpu.SemaphoreType.DMA,
        pltpu.SemaphoreType.DMA,
    ],
    compiler_params=pltpu.CompilerParams(use_tc_tiling_on_sc=False,
                                         needs_layout_passes=False),
)(_sc_body)


def kernel(cat_features, interaction_table):
    cat_flat = cat_features.reshape(-1)
    ii = jnp.asarray(_II)
    jj = jnp.asarray(_JJ)
    out = _sc_kernel(cat_flat, interaction_table, ii, jj)
    out5 = out.reshape(NPAIR, NTD, NW, 8, 128)
    return out5.transpose(2, 4, 0, 1, 3).reshape(BATCH, NPAIR, DM)


# speedup vs baseline: 1.0710x; 1.0162x over previous
import functools

import numpy as np
import jax
import jax.numpy as jnp
from jax import lax
from jax.experimental import pallas as pl
from jax.experimental.pallas import tpu as pltpu
from jax.experimental.pallas import tpu_sc as plsc

NCAT = 26
HASH = 100000
DM = 32
BATCH = 4096
NPAIR = NCAT * (NCAT - 1) // 2

_info = plsc.get_sparse_core_info()
_NC, _NS, _L = _info.num_cores, _info.num_subcores, _info.num_lanes
NW = _NC * _NS

BPW = BATCH // NW
PG = 5
NBLK = NPAIR // PG
CROWS = PG * BPW
NTD = DM // 8

IPAD = 328

_i_np, _j_np = np.triu_indices(NCAT, k=1)
_II = np.zeros((IPAD,), np.int32)
_JJ = np.zeros((IPAD,), np.int32)
_II[:NPAIR] = _i_np
_JJ[:NPAIR] = _j_np


def _mod_const(n, d):
    rinv = np.float32(1.0) / np.float32(d)
    q = (n.astype(jnp.float32) * rinv).astype(jnp.int32)
    r = n - q * d
    q = jnp.where(r >= d, q + 1, q)
    q = jnp.where(r < 0, q - 1, q)
    return n - q * d


def _sc_body(cat_hbm, tab_hbm, ii_hbm, jj_hbm, out_hbm,
             cat_v, ii_v, jj_v, idx0, idx1, idx2, rows0, rows1, rows2,
             stg_v, gsem, wsem):
    wid = lax.axis_index("s") * _NC + lax.axis_index("c")
    pltpu.sync_copy(cat_hbm.at[pl.ds(wid * (BPW * NCAT), BPW * NCAT)], cat_v)
    pltpu.sync_copy(ii_hbm, ii_v)
    pltpu.sync_copy(jj_hbm, jj_v)
    lanes = lax.iota(jnp.int32, _L)

    def comp_fire(blk, idx_v, rows_v):
        p0 = blk * PG

        def comp(pl_i, c):
            pvec = (p0 + pl_i) + lanes * 0
            ip = plsc.load_gather(ii_v, [pvec])
            jp = plsc.load_gather(jj_v, [pvec])
            for v in range(BPW // _L):
                b_l = v * _L + lanes
                base26 = b_l * NCAT
                ci = plsc.load_gather(cat_v, [base26 + ip])
                cj = plsc.load_gather(cat_v, [base26 + jp])
                h = _mod_const(ci * 17 + cj * 31, HASH)
                idx_v[pl.ds(pl_i * BPW + v * _L, _L)] = h
            return c
        lax.fori_loop(0, PG, comp, 0)

        def fire(si, c):
            pltpu.async_copy(tab_hbm.at[idx_v.at[pl.ds(si * 128, 128)]],
                             rows_v.at[pl.ds(si * 128, 128)], gsem)
            return c
        lax.fori_loop(0, PG, fire, 0)

    def wfire(blk):
        def per_p(pl_i, c):
            pltpu.async_copy(
                stg_v.at[pl.ds(pl_i * NTD, NTD), :, pl.ds(0, BPW)],
                out_hbm.at[pl.ds((blk * PG + pl_i) * NTD, NTD),
                           pl.ds(wid * 8, 8)], wsem)
            return c
        lax.fori_loop(0, PG, per_p, 0)

    def wdrain():
        pltpu.make_async_copy(out_hbm.at[pl.ds(0, PG * NTD), pl.ds(0, 8)],
                              stg_v.at[:, :, pl.ds(0, BPW)], wsem).wait()

    lanes_td = lax.shift_right_logical(lanes, 3)
    lanes_s = lanes & 7

    def process(blk, idx_v, rows_v):
        @pl.when(blk > 0)
        def _():
            wdrain()

        pltpu.make_async_copy(out_hbm.at[pl.ds(0, PG * NTD), pl.ds(0, 8)],
                              stg_v.at[:, :, pl.ds(0, BPW)], gsem).wait()

        def trans_p(pl_i, c):
            td0 = pl_i * NTD + lanes_td
            td1 = td0 + 2

            def trans_g(g, c2):
                base = pl_i * BPW + g * 16

                def load(j):
                    return (rows_v[base + j, pl.ds(0, _L)],
                            rows_v[base + j, pl.ds(_L, _L)])
                xs = [load(0), load(1), load(2), load(3)]
                for j in range(16):
                    if j + 4 < 16:
                        xs.append(load(j + 4))
                    x0, x1 = xs[j]
                    cvec = (g * 16 + j) + lanes * 0
                    plsc.store_scatter(stg_v, [td0, lanes_s, cvec], x0)
                    plsc.store_scatter(stg_v, [td1, lanes_s, cvec], x1)
                return c2
            lax.fori_loop(0, BPW // 16, trans_g, 0)
            return c
        lax.fori_loop(0, PG, trans_p, 0)

        wfire(blk)

    comp_fire(0, idx0, rows0)
    comp_fire(1, idx1, rows1)

    def step(t, carry):
        b = 3 * t
        comp_fire(b + 2, idx2, rows2)
        process(b, idx0, rows0)
        comp_fire(b + 3, idx0, rows0)
        process(b + 1, idx1, rows1)
        comp_fire(b + 4, idx1, rows1)
        process(b + 2, idx2, rows2)
        return carry

    lax.fori_loop(0, (NBLK - 2) // 3, step, 0)
    process(NBLK - 2, idx0, rows0)
    process(NBLK - 1, idx1, rows1)
    wdrain()


_mesh = plsc.VectorSubcoreMesh(core_axis_name="c", subcore_axis_name="s")

_sc_kernel = functools.partial(
    pl.kernel,
    mesh=_mesh,
    out_type=jax.ShapeDtypeStruct((NPAIR * NTD, NW * 8, 128), jnp.float32),
    scratch_types=[
        pltpu.VMEM((BPW * NCAT,), jnp.int32),
        pltpu.VMEM((IPAD,), jnp.int32),
        pltpu.VMEM((IPAD,), jnp.int32),
        pltpu.VMEM((CROWS,), jnp.int32),
        pltpu.VMEM((CROWS,), jnp.int32),
        pltpu.VMEM((CROWS,), jnp.int32),
        pltpu.VMEM((CROWS, DM), jnp.float32),
        pltpu.VMEM((CROWS, DM), jnp.float32),
        pltpu.VMEM((CROWS, DM), jnp.float32),
        pltpu.VMEM((PG * NTD, 8, BPW + 1), jnp.float32),
        pltpu.SemaphoreType.DMA,
        pltpu.SemaphoreType.DMA,
    ],
    compiler_params=pltpu.CompilerParams(use_tc_tiling_on_sc=False,
                                         needs_layout_passes=False),
)(_sc_body)


def kernel(cat_features, interaction_table):
    cat_flat = cat_features.reshape(-1)
    ii = jnp.asarray(_II)
    jj = jnp.asarray(_JJ)
    out = _sc_kernel(cat_flat, interaction_table, ii, jj)
    out5 = out.reshape(NPAIR, NTD, NW, 8, 128)
    return out5.transpose(2, 4, 0, 1, 3).reshape(BATCH, NPAIR, DM)
